# Initial kernel scaffold; baseline (speedup 1.0000x reference)
import jax
import jax.numpy as jnp
from jax.experimental import pallas as pl


def kernel(inputs, x, table):
    raise NotImplementedError("write your pallas kernel here")


# speedup vs baseline: 5.0289x; 5.0289x over previous
import jax
import jax.numpy as jnp
from jax.experimental import pallas as pl

_NUM_BLOCKS = 64


def _body(idx_ref, x_ref, table_ref, out_ref):
    idx = idx_ref[0, 0, :]
    iota = jax.lax.broadcasted_iota(jnp.int32, (32, idx.shape[0]), 0)
    oht = (idx[None, :] == iota).astype(jnp.float32)
    emb = jax.lax.dot_general(
        oht, table_ref[...], (((0,), (0,)), ((), ())),
        preferred_element_type=jnp.float32,
    )
    out_ref[...] = x_ref[...] + emb


def kernel(inputs, x, table):
    B, S, D = x.shape
    N = B * S
    R = N // _NUM_BLOCKS
    idx3 = inputs.reshape(_NUM_BLOCKS, 1, R)
    x2 = x.reshape(N, D)
    out2 = pl.pallas_call(
        _body,
        grid=(_NUM_BLOCKS,),
        in_specs=[
            pl.BlockSpec((1, 1, R), lambda i: (i, 0, 0)),
            pl.BlockSpec((R, D), lambda i: (i, 0)),
            pl.BlockSpec((32, D), lambda i: (0, 0)),
        ],
        out_specs=pl.BlockSpec((R, D), lambda i: (i, 0)),
        out_shape=jax.ShapeDtypeStruct((N, D), x.dtype),
    )(idx3, x2, table)
    return out2.reshape(B, S, D)
